# baseline (speedup 1.0000x reference)
import functools

import jax
import jax.numpy as jnp
from jax import lax
from jax.experimental import pallas as pl
from jax.experimental.pallas import tpu as pltpu
from jax.experimental.pallas import tpu_sc as plsc

E = 8
TOP_K = 2
D = 768
H_SHARED = 1536
H_ROUTED = 768
N_TOK = 2048
N_PAIR = N_TOK * TOP_K
BLK = 128
P_ROWS = N_PAIR + E * BLK
NB = P_ROWS // BLK
BT = 256
NW = 32
L = 16


def _k1_body(x_ref, temb_ref, wxT_ref, wtT_ref, bias_ref,
             sw1T_ref, sw3T_ref, sw2T_ref,
             shared_ref, topi_ref, gates_ref):
    xb = x_ref[...]
    xb16 = xb.astype(jnp.bfloat16)
    a = jnp.dot(xb16, sw1T_ref[...], preferred_element_type=jnp.float32)
    b = jnp.dot(xb16, sw3T_ref[...], preferred_element_type=jnp.float32)
    h = (a * jax.nn.sigmoid(a) * b).astype(jnp.bfloat16)
    shared_ref[...] = jnp.dot(h, sw2T_ref[...],
                              preferred_element_type=jnp.float32)
    logits = (jnp.dot(xb, wxT_ref[...], preferred_element_type=jnp.float32)
              + jnp.dot(temb_ref[...], wtT_ref[...],
                        preferred_element_type=jnp.float32))
    s = jax.nn.sigmoid(logits)
    sel = s + bias_ref[...]
    idx = jax.lax.broadcasted_iota(jnp.int32, sel.shape, 1)
    neg = jnp.float32(-jnp.inf)
    m1 = jnp.max(sel, axis=1, keepdims=True)
    i1 = jnp.min(jnp.where(sel == m1, idx, E), axis=1, keepdims=True)
    sel2 = jnp.where(idx == i1, neg, sel)
    m2 = jnp.max(sel2, axis=1, keepdims=True)
    i2 = jnp.min(jnp.where(sel2 == m2, idx, E), axis=1, keepdims=True)
    s1 = jnp.sum(jnp.where(idx == i1, s, 0.0), axis=1, keepdims=True)
    s2 = jnp.sum(jnp.where(idx == i2, s, 0.0), axis=1, keepdims=True)
    denom = s1 + s2
    ok = denom > 1e-9
    g1 = jnp.where(ok, s1 / (denom + 1e-9), 1.0 / TOP_K)
    g2 = jnp.where(ok, s2 / (denom + 1e-9), 1.0 / TOP_K)
    topi_ref[:, 0:1] = i1
    topi_ref[:, 1:2] = i2
    gates_ref[:, 0:1] = g1
    gates_ref[:, 1:2] = g2


def _router_shared(xf, t_emb, router_W, router_bias, sw1, sw3, sw2):
    wxT = router_W[:, :D].T
    wtT = router_W[:, D:].T
    grid = N_TOK // BT
    return pl.pallas_call(
        _k1_body,
        grid=(grid,),
        in_specs=[
            pl.BlockSpec((BT, D), lambda i: (i, 0)),
            pl.BlockSpec((1, D), lambda i: (0, 0)),
            pl.BlockSpec((D, E), lambda i: (0, 0)),
            pl.BlockSpec((D, E), lambda i: (0, 0)),
            pl.BlockSpec((1, E), lambda i: (0, 0)),
            pl.BlockSpec((D, H_SHARED), lambda i: (0, 0)),
            pl.BlockSpec((D, H_SHARED), lambda i: (0, 0)),
            pl.BlockSpec((H_SHARED, D), lambda i: (0, 0)),
        ],
        out_specs=[
            pl.BlockSpec((BT, D), lambda i: (i, 0)),
            pl.BlockSpec((BT, TOP_K), lambda i: (i, 0)),
            pl.BlockSpec((BT, TOP_K), lambda i: (i, 0)),
        ],
        out_shape=[
            jax.ShapeDtypeStruct((N_TOK, D), jnp.float32),
            jax.ShapeDtypeStruct((N_TOK, TOP_K), jnp.int32),
            jax.ShapeDtypeStruct((N_TOK, TOP_K), jnp.float32),
        ],
    )(xf, t_emb, wxT, wtT, router_bias.reshape(1, E),
      sw1.T.astype(jnp.bfloat16), sw3.T.astype(jnp.bfloat16),
      sw2.T.astype(jnp.bfloat16))


def _k1b_body(topi_ref, gates_ref, slots_ref, bem_ref, gb_ref,
              cnt_ref, base_ref):
    p = pl.program_id(0)
    i = pl.program_id(1)
    ti = topi_ref[...]
    i1 = ti[:, 0:1]
    i2 = ti[:, 1:2]
    idx8 = lax.broadcasted_iota(jnp.int32, (BT, E), 1)
    oh1 = (idx8 == i1).astype(jnp.float32)
    oh2 = (idx8 == i2).astype(jnp.float32)
    S = oh1 + oh2
    colsum = jnp.sum(S, axis=0, keepdims=True)

    @pl.when(jnp.logical_and(p == 0, i == 0))
    def _():
        cnt_ref[...] = jnp.zeros_like(cnt_ref)

    @pl.when(p == 0)
    def _():
        cnt_ref[...] = cnt_ref[...] + colsum

    @pl.when(jnp.logical_and(p == 1, i == 0))
    def _():
        cnt = cnt_ref[...].astype(jnp.int32)
        aligned = ((cnt + (BLK - 1)) >> 7) << 7
        er = lax.broadcasted_iota(jnp.int32, (E, E), 0)
        ec = lax.broadcasted_iota(jnp.int32, (E, E), 1)
        tri8 = (er < ec).astype(jnp.float32)
        base = jnp.dot(aligned.astype(jnp.float32), tri8,
                       preferred_element_type=jnp.float32)
        base_ref[...] = base
        cnt_ref[...] = jnp.zeros_like(cnt_ref)
        bs = lax.broadcasted_iota(jnp.int32, (BLK, E), 0) * BLK
        hits = (bs.astype(jnp.float32) >= base).astype(jnp.float32)
        bem_ref[...] = (jnp.sum(hits, axis=1, keepdims=True)
                        - 1.0).astype(jnp.int32)

    @pl.when(p == 1)
    def _():
        tr = lax.broadcasted_iota(jnp.int32, (BT, BT), 0)
        tc = lax.broadcasted_iota(jnp.int32, (BT, BT), 1)
        tril = (tc < tr).astype(jnp.float32)
        pe = jnp.dot(tril, S, preferred_element_type=jnp.float32)
        off = pe + base_ref[...] + cnt_ref[...]
        r1 = jnp.sum(jnp.where(oh1 > 0, off, 0.0), axis=1, keepdims=True)
        r2 = jnp.sum(jnp.where(oh2 > 0, off, 0.0), axis=1, keepdims=True)
        slots_ref[:, 0:1] = r1.astype(jnp.int32)
        slots_ref[:, 1:2] = r2.astype(jnp.int32)
        cnt_ref[...] = cnt_ref[...] + colsum
        g = gates_ref[...]
        gb_ref[:, 0:L] = jnp.broadcast_to(g[:, 0:1], (BT, L))
        gb_ref[:, L:2 * L] = jnp.broadcast_to(g[:, 1:2], (BT, L))


def _slot_assign(topi, gates):
    grid = N_TOK // BT
    return pl.pallas_call(
        _k1b_body,
        grid=(2, grid),
        in_specs=[
            pl.BlockSpec((BT, TOP_K), lambda p, i: (i, 0)),
            pl.BlockSpec((BT, TOP_K), lambda p, i: (i, 0)),
        ],
        out_specs=[
            pl.BlockSpec((BT, TOP_K), lambda p, i: (i, 0)),
            pl.BlockSpec((BLK, 1), lambda p, i: (0, 0)),
            pl.BlockSpec((BT, 2 * L), lambda p, i: (i, 0)),
        ],
        out_shape=[
            jax.ShapeDtypeStruct((N_TOK, TOP_K), jnp.int32),
            jax.ShapeDtypeStruct((BLK, 1), jnp.int32),
            jax.ShapeDtypeStruct((N_TOK, 2 * L), jnp.float32),
        ],
        scratch_shapes=[
            pltpu.VMEM((1, E), jnp.float32),
            pltpu.VMEM((1, E), jnp.float32),
        ],
    )(topi, gates)


_TPW = N_TOK // NW


def _k2_body(slots0_hbm, slots1_hbm, xf_hbm, xs_hbm,
             idx0_v, idx1_v, rows_v, sem0, sem1):
    wid = lax.axis_index("s") * 2 + lax.axis_index("c")
    base = wid * _TPW
    pltpu.sync_copy(slots0_hbm.at[pl.ds(base, _TPW)], idx0_v)
    pltpu.sync_copy(slots1_hbm.at[pl.ds(base, _TPW)], idx1_v)
    pltpu.sync_copy(xf_hbm.at[pl.ds(base, _TPW)], rows_v)
    cp0 = pltpu.async_copy(rows_v, xs_hbm.at[idx0_v], sem0)
    cp1 = pltpu.async_copy(rows_v, xs_hbm.at[idx1_v], sem1)
    cp0.wait()
    cp1.wait()


def _scatter_sc(slots0, slots1, xf):
    mesh = plsc.VectorSubcoreMesh(core_axis_name="c", subcore_axis_name="s")
    f = functools.partial(
        pl.kernel,
        out_type=jax.ShapeDtypeStruct((P_ROWS, D), jnp.float32),
        mesh=mesh,
        scratch_types=[
            pltpu.VMEM((_TPW,), jnp.int32),
            pltpu.VMEM((_TPW,), jnp.int32),
            pltpu.VMEM((_TPW, D), jnp.float32),
            pltpu.SemaphoreType.DMA,
            pltpu.SemaphoreType.DMA,
        ],
    )
    return f(_k2_body)(slots0, slots1, xf)


def _k3_body(bem_ref, xs_ref, ew1_ref, ew2_ref, ys_ref):
    h = jnp.dot(xs_ref[...].astype(jnp.bfloat16), ew1_ref[0],
                preferred_element_type=jnp.float32)
    h = h * 0.5 * (1.0 + jax.lax.erf(h * (2.0 ** -0.5)))
    ys_ref[...] = jnp.dot(h.astype(jnp.bfloat16), ew2_ref[0],
                          preferred_element_type=jnp.float32)


def _expert_ffn(xs, ew1, ew2, bem):
    grid_spec = pltpu.PrefetchScalarGridSpec(
        num_scalar_prefetch=1,
        grid=(NB,),
        in_specs=[
            pl.BlockSpec((BLK, D), lambda b, bem: (b, 0)),
            pl.BlockSpec((1, D, H_ROUTED), lambda b, bem: (bem[b], 0, 0)),
            pl.BlockSpec((1, H_ROUTED, D), lambda b, bem: (bem[b], 0, 0)),
        ],
        out_specs=pl.BlockSpec((BLK, D), lambda b, bem: (b, 0)),
    )
    return pl.pallas_call(
        _k3_body,
        grid_spec=grid_spec,
        out_shape=jax.ShapeDtypeStruct((P_ROWS, D), jnp.float32),
    )(bem, xs, ew1.astype(jnp.bfloat16), ew2.astype(jnp.bfloat16))


_CH = _TPW // 2


def _k4_body(slots0_hbm, slots1_hbm, gb_hbm, ys_hbm, shared_hbm, out_hbm,
             idx0_v, idx1_v, rows0_v, rows1_v, gb_v, sh_v, out_v,
             sem0, sem1):
    wid = lax.axis_index("s") * 2 + lax.axis_index("c")
    for half in range(2):
        tbase = wid * _TPW + half * _CH
        pltpu.sync_copy(slots0_hbm.at[pl.ds(tbase, _CH)], idx0_v)
        pltpu.sync_copy(slots1_hbm.at[pl.ds(tbase, _CH)], idx1_v)
        cp0 = pltpu.async_copy(ys_hbm.at[idx0_v], rows0_v, sem0)
        cp1 = pltpu.async_copy(ys_hbm.at[idx1_v], rows1_v, sem1)
        pltpu.sync_copy(gb_hbm.at[pl.ds(tbase, _CH)], gb_v)
        pltpu.sync_copy(shared_hbm.at[pl.ds(tbase, _CH)], sh_v)
        cp0.wait()
        cp1.wait()

        def body(t, carry):
            g0 = gb_v[t, pl.ds(0, L)]
            g1 = gb_v[t, pl.ds(L, L)]
            for c in range(D // L):
                cs = pl.ds(c * L, L)
                out_v[t, cs] = (sh_v[t, cs] + g0 * rows0_v[t, cs]
                                + g1 * rows1_v[t, cs]) * (1.0 / 3.0)
            return carry

        lax.fori_loop(0, _CH, body, 0)
        pltpu.sync_copy(out_v, out_hbm.at[pl.ds(tbase, _CH)])


def _combine_sc(slots0, slots1, gb, ys, shared):
    mesh = plsc.VectorSubcoreMesh(core_axis_name="c", subcore_axis_name="s")
    f = functools.partial(
        pl.kernel,
        out_type=jax.ShapeDtypeStruct((N_TOK, D), jnp.float32),
        mesh=mesh,
        scratch_types=[
            pltpu.VMEM((_CH,), jnp.int32),
            pltpu.VMEM((_CH,), jnp.int32),
            pltpu.VMEM((_CH, D), jnp.float32),
            pltpu.VMEM((_CH, D), jnp.float32),
            pltpu.VMEM((_CH, 2 * L), jnp.float32),
            pltpu.VMEM((_CH, D), jnp.float32),
            pltpu.VMEM((_CH, D), jnp.float32),
            pltpu.SemaphoreType.DMA,
            pltpu.SemaphoreType.DMA,
        ],
    )
    return f(_k4_body)(slots0, slots1, gb, ys, shared)


def kernel(x, t_emb, router_W, sw1, sw3, sw2, ew1, ew2, router_bias):
    B, T, C = x.shape
    xf = x.reshape(-1, C)

    shared, topi, gates = _router_shared(xf, t_emb, router_W, router_bias,
                                         sw1, sw3, sw2)
    slots, bem, gb = _slot_assign(topi, gates)
    slots0 = slots[:, 0]
    slots1 = slots[:, 1]

    xs = _scatter_sc(slots0, slots1, xf)
    ys = _expert_ffn(xs, ew1, ew2, bem[:NB, 0])
    out = _combine_sc(slots0, slots1, gb, ys, shared)
    return out.reshape(B, T, C)


# speedup vs baseline: 1.1944x; 1.1944x over previous
import functools

import jax
import jax.numpy as jnp
from jax import lax
from jax.experimental import pallas as pl
from jax.experimental.pallas import tpu as pltpu
from jax.experimental.pallas import tpu_sc as plsc

E = 8
TOP_K = 2
D = 768
H_SHARED = 1536
H_ROUTED = 768
N_TOK = 2048
N_PAIR = N_TOK * TOP_K
BLK = 128
P_ROWS = N_PAIR + E * BLK
NB = P_ROWS // BLK
BT = 256
NW = 32
L = 16

_NT = (((1,), (1,)), ((), ()))


def _k1_body(x_ref, temb_ref, rw_ref, bias_ref, sw1_ref, sw3_ref, sw2_ref,
             shared_ref, topi_ref, gates_ref, counts_ref, cnt_scr):
    i = pl.program_id(0)
    xb = x_ref[...]
    a = lax.dot_general(xb, sw1_ref[...], _NT,
                        preferred_element_type=jnp.float32)
    b = lax.dot_general(xb, sw3_ref[...], _NT,
                        preferred_element_type=jnp.float32)
    h = a * jax.nn.sigmoid(a) * b
    shared_ref[...] = lax.dot_general(h, sw2_ref[...], _NT,
                                      preferred_element_type=jnp.float32)
    wx = rw_ref[:, :D]
    wt = rw_ref[:, D:]
    logits = (lax.dot_general(xb, wx, _NT,
                              preferred_element_type=jnp.float32)
              + lax.dot_general(temb_ref[...], wt, _NT,
                                preferred_element_type=jnp.float32))
    s = jax.nn.sigmoid(logits)
    sel = s + bias_ref[...]
    idx = lax.broadcasted_iota(jnp.int32, sel.shape, 1)
    neg = jnp.float32(-jnp.inf)
    m1 = jnp.max(sel, axis=1, keepdims=True)
    i1 = jnp.min(jnp.where(sel == m1, idx, E), axis=1, keepdims=True)
    sel2 = jnp.where(idx == i1, neg, sel)
    m2 = jnp.max(sel2, axis=1, keepdims=True)
    i2 = jnp.min(jnp.where(sel2 == m2, idx, E), axis=1, keepdims=True)
    s1 = jnp.sum(jnp.where(idx == i1, s, 0.0), axis=1, keepdims=True)
    s2 = jnp.sum(jnp.where(idx == i2, s, 0.0), axis=1, keepdims=True)
    denom = s1 + s2
    ok = denom > 1e-9
    g1 = jnp.where(ok, s1 / (denom + 1e-9), 1.0 / TOP_K)
    g2 = jnp.where(ok, s2 / (denom + 1e-9), 1.0 / TOP_K)
    topi_ref[:, 0:1] = i1
    topi_ref[:, 1:2] = i2
    gates_ref[:, 0:1] = g1
    gates_ref[:, 1:2] = g2
    oh = ((idx == i1) | (idx == i2)).astype(jnp.float32)
    colsum = jnp.sum(oh, axis=0, keepdims=True)

    @pl.when(i == 0)
    def _():
        cnt_scr[...] = jnp.zeros_like(cnt_scr)

    cnt_scr[...] = cnt_scr[...] + colsum
    counts_ref[...] = cnt_scr[...]


def _router_shared(xf, t_emb, router_W, router_bias, sw1, sw3, sw2):
    grid = N_TOK // BT
    return pl.pallas_call(
        _k1_body,
        grid=(grid,),
        in_specs=[
            pl.BlockSpec((BT, D), lambda i: (i, 0)),
            pl.BlockSpec((1, D), lambda i: (0, 0)),
            pl.BlockSpec((E, 2 * D), lambda i: (0, 0)),
            pl.BlockSpec((1, E), lambda i: (0, 0)),
            pl.BlockSpec((H_SHARED, D), lambda i: (0, 0)),
            pl.BlockSpec((H_SHARED, D), lambda i: (0, 0)),
            pl.BlockSpec((D, H_SHARED), lambda i: (0, 0)),
        ],
        out_specs=[
            pl.BlockSpec((BT, D), lambda i: (i, 0)),
            pl.BlockSpec((BT, TOP_K), lambda i: (i, 0)),
            pl.BlockSpec((BT, TOP_K), lambda i: (i, 0)),
            pl.BlockSpec((1, E), lambda i: (0, 0)),
        ],
        out_shape=[
            jax.ShapeDtypeStruct((N_TOK, D), jnp.float32),
            jax.ShapeDtypeStruct((N_TOK, TOP_K), jnp.int32),
            jax.ShapeDtypeStruct((N_TOK, TOP_K), jnp.float32),
            jax.ShapeDtypeStruct((1, E), jnp.float32),
        ],
        scratch_shapes=[pltpu.VMEM((1, E), jnp.float32)],
    )(xf, t_emb, router_W, router_bias.reshape(1, E), sw1, sw3, sw2)


def _k1b_body(topi_ref, counts_ref, slots0_ref, slots1_ref, bem_ref,
              cnt_ref, base_ref):
    i = pl.program_id(0)
    ti = topi_ref[...]
    i1 = ti[:, 0:1]
    i2 = ti[:, 1:2]
    idx8 = lax.broadcasted_iota(jnp.int32, (BT, E), 1)
    oh1 = (idx8 == i1).astype(jnp.float32)
    oh2 = (idx8 == i2).astype(jnp.float32)
    S = oh1 + oh2

    @pl.when(i == 0)
    def _():
        cnt = counts_ref[...].astype(jnp.int32)
        aligned = ((cnt + (BLK - 1)) >> 7) << 7
        er = lax.broadcasted_iota(jnp.int32, (E, E), 0)
        ec = lax.broadcasted_iota(jnp.int32, (E, E), 1)
        tri8 = (er < ec).astype(jnp.float32)
        base = jnp.dot(aligned.astype(jnp.float32), tri8,
                       preferred_element_type=jnp.float32)
        base_ref[...] = base
        cnt_ref[...] = jnp.zeros_like(cnt_ref)
        bs = lax.broadcasted_iota(jnp.int32, (BLK, E), 0) * BLK
        hits = (bs.astype(jnp.float32) >= base).astype(jnp.float32)
        bem_ref[...] = (jnp.sum(hits, axis=1, keepdims=True)
                        - 1.0).astype(jnp.int32)

    tr = lax.broadcasted_iota(jnp.int32, (BT, BT), 0)
    tc = lax.broadcasted_iota(jnp.int32, (BT, BT), 1)
    tril = (tc < tr).astype(jnp.float32)
    pe = jnp.dot(tril, S, preferred_element_type=jnp.float32)
    off = pe + base_ref[...] + cnt_ref[...]
    r1 = jnp.sum(jnp.where(oh1 > 0, off, 0.0), axis=1, keepdims=True)
    r2 = jnp.sum(jnp.where(oh2 > 0, off, 0.0), axis=1, keepdims=True)
    slots0_ref[...] = r1.astype(jnp.int32)
    slots1_ref[...] = r2.astype(jnp.int32)
    cnt_ref[...] = cnt_ref[...] + jnp.sum(S, axis=0, keepdims=True)


def _slot_assign(topi, counts):
    grid = N_TOK // BT
    return pl.pallas_call(
        _k1b_body,
        grid=(grid,),
        in_specs=[
            pl.BlockSpec((BT, TOP_K), lambda i: (i, 0)),
            pl.BlockSpec((1, E), lambda i: (0, 0)),
        ],
        out_specs=[
            pl.BlockSpec((BT, 1), lambda i: (i, 0)),
            pl.BlockSpec((BT, 1), lambda i: (i, 0)),
            pl.BlockSpec((BLK, 1), lambda i: (0, 0)),
        ],
        out_shape=[
            jax.ShapeDtypeStruct((N_TOK, 1), jnp.int32),
            jax.ShapeDtypeStruct((N_TOK, 1), jnp.int32),
            jax.ShapeDtypeStruct((BLK, 1), jnp.int32),
        ],
        scratch_shapes=[
            pltpu.VMEM((1, E), jnp.float32),
            pltpu.VMEM((1, E), jnp.float32),
        ],
    )(topi, counts)


_TPW = N_TOK // NW


def _k2_body(slots0_hbm, slots1_hbm, xf_hbm, xs_hbm,
             idx0_v, idx1_v, rows_v, sem0, sem1):
    wid = lax.axis_index("s") * 2 + lax.axis_index("c")
    base = wid * _TPW
    pltpu.sync_copy(slots0_hbm.at[pl.ds(base, _TPW)], idx0_v)
    pltpu.sync_copy(slots1_hbm.at[pl.ds(base, _TPW)], idx1_v)
    pltpu.sync_copy(xf_hbm.at[pl.ds(base, _TPW)], rows_v)
    cp0 = pltpu.async_copy(rows_v, xs_hbm.at[idx0_v], sem0)
    cp1 = pltpu.async_copy(rows_v, xs_hbm.at[idx1_v], sem1)
    cp0.wait()
    cp1.wait()


def _scatter_sc(slots0, slots1, xf):
    mesh = plsc.VectorSubcoreMesh(core_axis_name="c", subcore_axis_name="s")
    f = functools.partial(
        pl.kernel,
        out_type=jax.ShapeDtypeStruct((P_ROWS, D), jnp.float32),
        mesh=mesh,
        scratch_types=[
            pltpu.VMEM((_TPW,), jnp.int32),
            pltpu.VMEM((_TPW,), jnp.int32),
            pltpu.VMEM((_TPW, D), jnp.float32),
            pltpu.SemaphoreType.DMA,
            pltpu.SemaphoreType.DMA,
        ],
    )
    return f(_k2_body)(slots0, slots1, xf)


def _k3_body(bem_ref, xs_ref, ew1_ref, ew2_ref, ys_ref):
    h = jnp.dot(xs_ref[...], ew1_ref[0],
                preferred_element_type=jnp.float32)
    h = h * 0.5 * (1.0 + jax.lax.erf(h * (2.0 ** -0.5)))
    ys_ref[...] = jnp.dot(h, ew2_ref[0],
                          preferred_element_type=jnp.float32)


def _expert_ffn(xs, ew1, ew2, bem):
    grid_spec = pltpu.PrefetchScalarGridSpec(
        num_scalar_prefetch=1,
        grid=(NB,),
        in_specs=[
            pl.BlockSpec((BLK, D), lambda b, bem: (b, 0)),
            pl.BlockSpec((1, D, H_ROUTED), lambda b, bem: (bem[b], 0, 0)),
            pl.BlockSpec((1, H_ROUTED, D), lambda b, bem: (bem[b], 0, 0)),
        ],
        out_specs=pl.BlockSpec((BLK, D), lambda b, bem: (b, 0)),
    )
    return pl.pallas_call(
        _k3_body,
        grid_spec=grid_spec,
        out_shape=jax.ShapeDtypeStruct((P_ROWS, D), jnp.float32),
    )(bem, xs, ew1, ew2)


def _k4_body(slots0_hbm, slots1_hbm, ys_hbm, rows0_hbm, rows1_hbm,
             idx0_v, idx1_v, rows0_v, rows1_v, sem0, sem1):
    wid = lax.axis_index("s") * 2 + lax.axis_index("c")
    base = wid * _TPW
    pltpu.sync_copy(slots0_hbm.at[pl.ds(base, _TPW)], idx0_v)
    pltpu.sync_copy(slots1_hbm.at[pl.ds(base, _TPW)], idx1_v)
    cp0 = pltpu.async_copy(ys_hbm.at[idx0_v], rows0_v, sem0)
    cp1 = pltpu.async_copy(ys_hbm.at[idx1_v], rows1_v, sem1)
    cp0.wait()
    pltpu.sync_copy(rows0_v, rows0_hbm.at[pl.ds(base, _TPW)])
    cp1.wait()
    pltpu.sync_copy(rows1_v, rows1_hbm.at[pl.ds(base, _TPW)])


def _gather_sc(slots0, slots1, ys):
    mesh = plsc.VectorSubcoreMesh(core_axis_name="c", subcore_axis_name="s")
    f = functools.partial(
        pl.kernel,
        out_type=[
            jax.ShapeDtypeStruct((N_TOK, D), jnp.float32),
            jax.ShapeDtypeStruct((N_TOK, D), jnp.float32),
        ],
        mesh=mesh,
        scratch_types=[
            pltpu.VMEM((_TPW,), jnp.int32),
            pltpu.VMEM((_TPW,), jnp.int32),
            pltpu.VMEM((_TPW, D), jnp.float32),
            pltpu.VMEM((_TPW, D), jnp.float32),
            pltpu.SemaphoreType.DMA,
            pltpu.SemaphoreType.DMA,
        ],
    )
    return f(_k4_body)(slots0, slots1, ys)


def _k5_body(shared_ref, rows0_ref, rows1_ref, gates_ref, out_ref):
    g = gates_ref[...]
    out_ref[...] = (shared_ref[...] + g[:, 0:1] * rows0_ref[...]
                    + g[:, 1:2] * rows1_ref[...]) * (1.0 / (1 + TOP_K))


def _combine_tc(shared, rows0, rows1, gates):
    grid = N_TOK // BT
    return pl.pallas_call(
        _k5_body,
        grid=(grid,),
        in_specs=[
            pl.BlockSpec((BT, D), lambda i: (i, 0)),
            pl.BlockSpec((BT, D), lambda i: (i, 0)),
            pl.BlockSpec((BT, D), lambda i: (i, 0)),
            pl.BlockSpec((BT, TOP_K), lambda i: (i, 0)),
        ],
        out_specs=pl.BlockSpec((BT, D), lambda i: (i, 0)),
        out_shape=jax.ShapeDtypeStruct((N_TOK, D), jnp.float32),
    )(shared, rows0, rows1, gates)


def kernel(x, t_emb, router_W, sw1, sw3, sw2, ew1, ew2, router_bias):
    B, T, C = x.shape
    xf = x.reshape(-1, C)

    shared, topi, gates, counts = _router_shared(
        xf, t_emb, router_W, router_bias, sw1, sw3, sw2)
    slots0, slots1, bem = _slot_assign(topi, counts)
    slots0 = slots0.reshape(N_TOK)
    slots1 = slots1.reshape(N_TOK)

    xs = _scatter_sc(slots0, slots1, xf)
    ys = _expert_ffn(xs, ew1, ew2, bem[:NB, 0])
    rows0, rows1 = _gather_sc(slots0, slots1, ys)
    out = _combine_tc(shared, rows0, rows1, gates)
    return out.reshape(B, T, C)


# speedup vs baseline: 1.2499x; 1.0465x over previous
import functools

import jax
import jax.numpy as jnp
from jax import lax
from jax.experimental import pallas as pl
from jax.experimental.pallas import tpu as pltpu
from jax.experimental.pallas import tpu_sc as plsc

E = 8
TOP_K = 2
D = 768
H_SHARED = 1536
H_ROUTED = 768
N_TOK = 2048
N_PAIR = N_TOK * TOP_K
BLK = 128
P_ROWS = N_PAIR + E * BLK
NB = P_ROWS // BLK
BT = 256
NW = 32
L = 16

_NT = (((1,), (1,)), ((), ()))


def _k0_body(x_ref, temb_ref, rw_ref, bias_ref,
             topi_ref, gates_ref, counts_ref):
    xb = x_ref[...]
    wx = rw_ref[:, :D]
    wt = rw_ref[:, D:]
    logits = (lax.dot_general(xb, wx, _NT,
                              preferred_element_type=jnp.float32)
              + lax.dot_general(temb_ref[...], wt, _NT,
                                preferred_element_type=jnp.float32))
    s = jax.nn.sigmoid(logits)
    sel = s + bias_ref[...]
    idx = lax.broadcasted_iota(jnp.int32, sel.shape, 1)
    neg = jnp.float32(-jnp.inf)
    m1 = jnp.max(sel, axis=1, keepdims=True)
    i1 = jnp.min(jnp.where(sel == m1, idx, E), axis=1, keepdims=True)
    sel2 = jnp.where(idx == i1, neg, sel)
    m2 = jnp.max(sel2, axis=1, keepdims=True)
    i2 = jnp.min(jnp.where(sel2 == m2, idx, E), axis=1, keepdims=True)
    s1 = jnp.sum(jnp.where(idx == i1, s, 0.0), axis=1, keepdims=True)
    s2 = jnp.sum(jnp.where(idx == i2, s, 0.0), axis=1, keepdims=True)
    denom = s1 + s2
    ok = denom > 1e-9
    g1 = jnp.where(ok, s1 / (denom + 1e-9), 1.0 / TOP_K)
    g2 = jnp.where(ok, s2 / (denom + 1e-9), 1.0 / TOP_K)
    topi_ref[:, 0:1] = i1
    topi_ref[:, 1:2] = i2
    gates_ref[:, 0:1] = g1
    gates_ref[:, 1:2] = g2
    oh = ((idx == i1) | (idx == i2)).astype(jnp.float32)
    counts_ref[...] = jnp.sum(oh, axis=0, keepdims=True)


def _router(xf, t_emb, router_W, router_bias):
    return pl.pallas_call(
        _k0_body,
        grid=(1,),
        in_specs=[
            pl.BlockSpec((N_TOK, D), lambda i: (0, 0)),
            pl.BlockSpec((1, D), lambda i: (0, 0)),
            pl.BlockSpec((E, 2 * D), lambda i: (0, 0)),
            pl.BlockSpec((1, E), lambda i: (0, 0)),
        ],
        out_specs=[
            pl.BlockSpec((N_TOK, TOP_K), lambda i: (0, 0)),
            pl.BlockSpec((N_TOK, TOP_K), lambda i: (0, 0)),
            pl.BlockSpec((1, E), lambda i: (0, 0)),
        ],
        out_shape=[
            jax.ShapeDtypeStruct((N_TOK, TOP_K), jnp.int32),
            jax.ShapeDtypeStruct((N_TOK, TOP_K), jnp.float32),
            jax.ShapeDtypeStruct((1, E), jnp.float32),
        ],
    )(xf, t_emb, router_W, router_bias.reshape(1, E))


def _k1_body(x_ref, sw1_ref, sw3_ref, sw2_ref, shared_ref):
    xb = x_ref[...]
    a = lax.dot_general(xb, sw1_ref[...], _NT,
                        preferred_element_type=jnp.float32)
    b = lax.dot_general(xb, sw3_ref[...], _NT,
                        preferred_element_type=jnp.float32)
    h = a * jax.nn.sigmoid(a) * b
    shared_ref[...] = lax.dot_general(h, sw2_ref[...], _NT,
                                      preferred_element_type=jnp.float32)


def _shared_expert(xf, sw1, sw3, sw2):
    grid = N_TOK // BT
    return pl.pallas_call(
        _k1_body,
        grid=(grid,),
        in_specs=[
            pl.BlockSpec((BT, D), lambda i: (i, 0)),
            pl.BlockSpec((H_SHARED, D), lambda i: (0, 0)),
            pl.BlockSpec((H_SHARED, D), lambda i: (0, 0)),
            pl.BlockSpec((D, H_SHARED), lambda i: (0, 0)),
        ],
        out_specs=pl.BlockSpec((BT, D), lambda i: (i, 0)),
        out_shape=jax.ShapeDtypeStruct((N_TOK, D), jnp.float32),
    )(xf, sw1, sw3, sw2)


def _k1b_body(topi_ref, counts_ref, slots0_ref, slots1_ref, bem_ref,
              cnt_ref, base_ref):
    i = pl.program_id(0)
    ti = topi_ref[...]
    i1 = ti[:, 0:1]
    i2 = ti[:, 1:2]
    idx8 = lax.broadcasted_iota(jnp.int32, (BT, E), 1)
    oh1 = (idx8 == i1).astype(jnp.float32)
    oh2 = (idx8 == i2).astype(jnp.float32)
    S = oh1 + oh2

    @pl.when(i == 0)
    def _():
        cnt = counts_ref[...].astype(jnp.int32)
        aligned = ((cnt + (BLK - 1)) >> 7) << 7
        er = lax.broadcasted_iota(jnp.int32, (E, E), 0)
        ec = lax.broadcasted_iota(jnp.int32, (E, E), 1)
        tri8 = (er < ec).astype(jnp.float32)
        base = jnp.dot(aligned.astype(jnp.float32), tri8,
                       preferred_element_type=jnp.float32)
        base_ref[...] = base
        cnt_ref[...] = jnp.zeros_like(cnt_ref)
        bs = lax.broadcasted_iota(jnp.int32, (BLK, E), 0) * BLK
        hits = (bs.astype(jnp.float32) >= base).astype(jnp.float32)
        bem_ref[...] = (jnp.sum(hits, axis=1, keepdims=True)
                        - 1.0).astype(jnp.int32)

    tr = lax.broadcasted_iota(jnp.int32, (BT, BT), 0)
    tc = lax.broadcasted_iota(jnp.int32, (BT, BT), 1)
    tril = (tc < tr).astype(jnp.float32)
    pe = jnp.dot(tril, S, preferred_element_type=jnp.float32)
    off = pe + base_ref[...] + cnt_ref[...]
    r1 = jnp.sum(jnp.where(oh1 > 0, off, 0.0), axis=1, keepdims=True)
    r2 = jnp.sum(jnp.where(oh2 > 0, off, 0.0), axis=1, keepdims=True)
    slots0_ref[...] = r1.astype(jnp.int32)
    slots1_ref[...] = r2.astype(jnp.int32)
    cnt_ref[...] = cnt_ref[...] + jnp.sum(S, axis=0, keepdims=True)


def _slot_assign(topi, counts):
    grid = N_TOK // BT
    return pl.pallas_call(
        _k1b_body,
        grid=(grid,),
        in_specs=[
            pl.BlockSpec((BT, TOP_K), lambda i: (i, 0)),
            pl.BlockSpec((1, E), lambda i: (0, 0)),
        ],
        out_specs=[
            pl.BlockSpec((BT, 1), lambda i: (i, 0)),
            pl.BlockSpec((BT, 1), lambda i: (i, 0)),
            pl.BlockSpec((BLK, 1), lambda i: (0, 0)),
        ],
        out_shape=[
            jax.ShapeDtypeStruct((N_TOK, 1), jnp.int32),
            jax.ShapeDtypeStruct((N_TOK, 1), jnp.int32),
            jax.ShapeDtypeStruct((BLK, 1), jnp.int32),
        ],
        scratch_shapes=[
            pltpu.VMEM((1, E), jnp.float32),
            pltpu.VMEM((1, E), jnp.float32),
        ],
    )(topi, counts)


_TPW = N_TOK // NW


def _k2_body(slots0_hbm, slots1_hbm, xf_hbm, xs_hbm,
             idx0_v, idx1_v, rows_v, sem0, sem1):
    wid = lax.axis_index("s") * 2 + lax.axis_index("c")
    base = wid * _TPW
    pltpu.sync_copy(slots0_hbm.at[pl.ds(base, _TPW)], idx0_v)
    pltpu.sync_copy(slots1_hbm.at[pl.ds(base, _TPW)], idx1_v)
    pltpu.sync_copy(xf_hbm.at[pl.ds(base, _TPW)], rows_v)
    cp0 = pltpu.async_copy(rows_v, xs_hbm.at[idx0_v], sem0)
    cp1 = pltpu.async_copy(rows_v, xs_hbm.at[idx1_v], sem1)
    cp0.wait()
    cp1.wait()


def _scatter_sc(slots0, slots1, xf):
    mesh = plsc.VectorSubcoreMesh(core_axis_name="c", subcore_axis_name="s")
    f = functools.partial(
        pl.kernel,
        out_type=jax.ShapeDtypeStruct((P_ROWS, D), jnp.float32),
        mesh=mesh,
        scratch_types=[
            pltpu.VMEM((_TPW,), jnp.int32),
            pltpu.VMEM((_TPW,), jnp.int32),
            pltpu.VMEM((_TPW, D), jnp.float32),
            pltpu.SemaphoreType.DMA,
            pltpu.SemaphoreType.DMA,
        ],
    )
    return f(_k2_body)(slots0, slots1, xf)


def _k3_body(bem_ref, xs_ref, ew1_ref, ew2_ref, ys_ref):
    h = jnp.dot(xs_ref[...], ew1_ref[0],
                preferred_element_type=jnp.float32)
    h = h * 0.5 * (1.0 + jax.lax.erf(h * (2.0 ** -0.5)))
    ys_ref[...] = jnp.dot(h, ew2_ref[0],
                          preferred_element_type=jnp.float32)


def _expert_ffn(xs, ew1, ew2, bem):
    grid_spec = pltpu.PrefetchScalarGridSpec(
        num_scalar_prefetch=1,
        grid=(NB,),
        in_specs=[
            pl.BlockSpec((BLK, D), lambda b, bem: (b, 0)),
            pl.BlockSpec((1, D, H_ROUTED), lambda b, bem: (bem[b], 0, 0)),
            pl.BlockSpec((1, H_ROUTED, D), lambda b, bem: (bem[b], 0, 0)),
        ],
        out_specs=pl.BlockSpec((BLK, D), lambda b, bem: (b, 0)),
    )
    return pl.pallas_call(
        _k3_body,
        grid_spec=grid_spec,
        out_shape=jax.ShapeDtypeStruct((P_ROWS, D), jnp.float32),
    )(bem, xs, ew1, ew2)


def _k4_body(slots0_hbm, slots1_hbm, ys_hbm, rows0_hbm, rows1_hbm,
             idx0_v, idx1_v, rows0_v, rows1_v, sem0, sem1):
    wid = lax.axis_index("s") * 2 + lax.axis_index("c")
    base = wid * _TPW
    pltpu.sync_copy(slots0_hbm.at[pl.ds(base, _TPW)], idx0_v)
    pltpu.sync_copy(slots1_hbm.at[pl.ds(base, _TPW)], idx1_v)
    cp0 = pltpu.async_copy(ys_hbm.at[idx0_v], rows0_v, sem0)
    cp1 = pltpu.async_copy(ys_hbm.at[idx1_v], rows1_v, sem1)
    cp0.wait()
    pltpu.sync_copy(rows0_v, rows0_hbm.at[pl.ds(base, _TPW)])
    cp1.wait()
    pltpu.sync_copy(rows1_v, rows1_hbm.at[pl.ds(base, _TPW)])


def _gather_sc(slots0, slots1, ys):
    mesh = plsc.VectorSubcoreMesh(core_axis_name="c", subcore_axis_name="s")
    f = functools.partial(
        pl.kernel,
        out_type=[
            jax.ShapeDtypeStruct((N_TOK, D), jnp.float32),
            jax.ShapeDtypeStruct((N_TOK, D), jnp.float32),
        ],
        mesh=mesh,
        scratch_types=[
            pltpu.VMEM((_TPW,), jnp.int32),
            pltpu.VMEM((_TPW,), jnp.int32),
            pltpu.VMEM((_TPW, D), jnp.float32),
            pltpu.VMEM((_TPW, D), jnp.float32),
            pltpu.SemaphoreType.DMA,
            pltpu.SemaphoreType.DMA,
        ],
    )
    return f(_k4_body)(slots0, slots1, ys)


def _k5_body(shared_ref, rows0_ref, rows1_ref, gates_ref, out_ref):
    g = gates_ref[...]
    out_ref[...] = (shared_ref[...] + g[:, 0:1] * rows0_ref[...]
                    + g[:, 1:2] * rows1_ref[...]) * (1.0 / (1 + TOP_K))


def _combine_tc(shared, rows0, rows1, gates):
    grid = N_TOK // BT
    return pl.pallas_call(
        _k5_body,
        grid=(grid,),
        in_specs=[
            pl.BlockSpec((BT, D), lambda i: (i, 0)),
            pl.BlockSpec((BT, D), lambda i: (i, 0)),
            pl.BlockSpec((BT, D), lambda i: (i, 0)),
            pl.BlockSpec((BT, TOP_K), lambda i: (i, 0)),
        ],
        out_specs=pl.BlockSpec((BT, D), lambda i: (i, 0)),
        out_shape=jax.ShapeDtypeStruct((N_TOK, D), jnp.float32),
    )(shared, rows0, rows1, gates)


def kernel(x, t_emb, router_W, sw1, sw3, sw2, ew1, ew2, router_bias):
    B, T, C = x.shape
    xf = x.reshape(-1, C)

    topi, gates, counts = _router(xf, t_emb, router_W, router_bias)
    slots0, slots1, bem = _slot_assign(topi, counts)
    slots0 = slots0.reshape(N_TOK)
    slots1 = slots1.reshape(N_TOK)

    xs = _scatter_sc(slots0, slots1, xf)
    shared = _shared_expert(xf, sw1, sw3, sw2)
    ys = _expert_ffn(xs, ew1, ew2, bem[:NB, 0])
    rows0, rows1 = _gather_sc(slots0, slots1, ys)
    out = _combine_tc(shared, rows0, rows1, gates)
    return out.reshape(B, T, C)


# speedup vs baseline: 1.3401x; 1.0721x over previous
import functools

import jax
import jax.numpy as jnp
from jax import lax
from jax.experimental import pallas as pl
from jax.experimental.pallas import tpu as pltpu
from jax.experimental.pallas import tpu_sc as plsc

E = 8
TOP_K = 2
D = 768
H_SHARED = 1536
H_ROUTED = 768
N_TOK = 2048
N_PAIR = N_TOK * TOP_K
BLK = 256
BLK_SH = BLK.bit_length() - 1
P_ROWS = N_PAIR + E * BLK
NB = P_ROWS // BLK
BT = 256
NW = 32
L = 16

_NT = (((1,), (1,)), ((), ()))


def _k0_body(x_ref, temb_ref, rw_ref, bias_ref,
             topi_ref, gates_ref, counts_ref):
    xb = x_ref[...]
    wx = rw_ref[:, :D]
    wt = rw_ref[:, D:]
    logits = (lax.dot_general(xb, wx, _NT,
                              preferred_element_type=jnp.float32)
              + lax.dot_general(temb_ref[...], wt, _NT,
                                preferred_element_type=jnp.float32))
    s = jax.nn.sigmoid(logits)
    sel = s + bias_ref[...]
    idx = lax.broadcasted_iota(jnp.int32, sel.shape, 1)
    neg = jnp.float32(-jnp.inf)
    m1 = jnp.max(sel, axis=1, keepdims=True)
    i1 = jnp.min(jnp.where(sel == m1, idx, E), axis=1, keepdims=True)
    sel2 = jnp.where(idx == i1, neg, sel)
    m2 = jnp.max(sel2, axis=1, keepdims=True)
    i2 = jnp.min(jnp.where(sel2 == m2, idx, E), axis=1, keepdims=True)
    s1 = jnp.sum(jnp.where(idx == i1, s, 0.0), axis=1, keepdims=True)
    s2 = jnp.sum(jnp.where(idx == i2, s, 0.0), axis=1, keepdims=True)
    denom = s1 + s2
    ok = denom > 1e-9
    g1 = jnp.where(ok, s1 / (denom + 1e-9), 1.0 / TOP_K)
    g2 = jnp.where(ok, s2 / (denom + 1e-9), 1.0 / TOP_K)
    topi_ref[:, 0:1] = i1
    topi_ref[:, 1:2] = i2
    gates_ref[:, 0:1] = g1
    gates_ref[:, 1:2] = g2
    oh = ((idx == i1) | (idx == i2)).astype(jnp.float32)
    counts_ref[...] = jnp.sum(oh, axis=0, keepdims=True)


def _router(xf, t_emb, router_W, router_bias):
    return pl.pallas_call(
        _k0_body,
        grid=(1,),
        in_specs=[
            pl.BlockSpec((N_TOK, D), lambda i: (0, 0)),
            pl.BlockSpec((1, D), lambda i: (0, 0)),
            pl.BlockSpec((E, 2 * D), lambda i: (0, 0)),
            pl.BlockSpec((1, E), lambda i: (0, 0)),
        ],
        out_specs=[
            pl.BlockSpec((N_TOK, TOP_K), lambda i: (0, 0)),
            pl.BlockSpec((N_TOK, TOP_K), lambda i: (0, 0)),
            pl.BlockSpec((1, E), lambda i: (0, 0)),
        ],
        out_shape=[
            jax.ShapeDtypeStruct((N_TOK, TOP_K), jnp.int32),
            jax.ShapeDtypeStruct((N_TOK, TOP_K), jnp.float32),
            jax.ShapeDtypeStruct((1, E), jnp.float32),
        ],
    )(xf, t_emb, router_W, router_bias.reshape(1, E))


def _k1_body(x_ref, sw1_ref, sw3_ref, sw2_ref, shared_ref):
    xb = x_ref[...]
    a = lax.dot_general(xb, sw1_ref[...], _NT,
                        preferred_element_type=jnp.float32)
    b = lax.dot_general(xb, sw3_ref[...], _NT,
                        preferred_element_type=jnp.float32)
    h = a * jax.nn.sigmoid(a) * b
    shared_ref[...] = lax.dot_general(h, sw2_ref[...], _NT,
                                      preferred_element_type=jnp.float32)


def _shared_expert(xf, sw1, sw3, sw2):
    grid = N_TOK // BT
    return pl.pallas_call(
        _k1_body,
        grid=(grid,),
        in_specs=[
            pl.BlockSpec((BT, D), lambda i: (i, 0)),
            pl.BlockSpec((H_SHARED, D), lambda i: (0, 0)),
            pl.BlockSpec((H_SHARED, D), lambda i: (0, 0)),
            pl.BlockSpec((D, H_SHARED), lambda i: (0, 0)),
        ],
        out_specs=pl.BlockSpec((BT, D), lambda i: (i, 0)),
        out_shape=jax.ShapeDtypeStruct((N_TOK, D), jnp.float32),
    )(xf, sw1, sw3, sw2)


def _k1b_body(topi_ref, counts_ref, slots0_ref, slots1_ref, bem_ref,
              cnt_ref, base_ref):
    i = pl.program_id(0)
    ti = topi_ref[...]
    i1 = ti[:, 0:1]
    i2 = ti[:, 1:2]
    idx8 = lax.broadcasted_iota(jnp.int32, (BT, E), 1)
    oh1 = (idx8 == i1).astype(jnp.float32)
    oh2 = (idx8 == i2).astype(jnp.float32)
    S = oh1 + oh2

    @pl.when(i == 0)
    def _():
        cnt = counts_ref[...].astype(jnp.int32)
        aligned = ((cnt + (BLK - 1)) >> BLK_SH) << BLK_SH
        er = lax.broadcasted_iota(jnp.int32, (E, E), 0)
        ec = lax.broadcasted_iota(jnp.int32, (E, E), 1)
        tri8 = (er < ec).astype(jnp.float32)
        base = jnp.dot(aligned.astype(jnp.float32), tri8,
                       preferred_element_type=jnp.float32)
        base_ref[...] = base
        cnt_ref[...] = jnp.zeros_like(cnt_ref)
        bs = lax.broadcasted_iota(jnp.int32, (BLK, E), 0) * BLK
        hits = (bs.astype(jnp.float32) >= base).astype(jnp.float32)
        bem_ref[...] = (jnp.sum(hits, axis=1, keepdims=True)
                        - 1.0).astype(jnp.int32)

    tr = lax.broadcasted_iota(jnp.int32, (BT, BT), 0)
    tc = lax.broadcasted_iota(jnp.int32, (BT, BT), 1)
    tril = (tc < tr).astype(jnp.float32)
    pe = jnp.dot(tril, S, preferred_element_type=jnp.float32)
    off = pe + base_ref[...] + cnt_ref[...]
    r1 = jnp.sum(jnp.where(oh1 > 0, off, 0.0), axis=1, keepdims=True)
    r2 = jnp.sum(jnp.where(oh2 > 0, off, 0.0), axis=1, keepdims=True)
    slots0_ref[...] = r1.astype(jnp.int32)
    slots1_ref[...] = r2.astype(jnp.int32)
    cnt_ref[...] = cnt_ref[...] + jnp.sum(S, axis=0, keepdims=True)


def _slot_assign(topi, counts):
    grid = N_TOK // BT
    return pl.pallas_call(
        _k1b_body,
        grid=(grid,),
        in_specs=[
            pl.BlockSpec((BT, TOP_K), lambda i: (i, 0)),
            pl.BlockSpec((1, E), lambda i: (0, 0)),
        ],
        out_specs=[
            pl.BlockSpec((BT, 1), lambda i: (i, 0)),
            pl.BlockSpec((BT, 1), lambda i: (i, 0)),
            pl.BlockSpec((BLK, 1), lambda i: (0, 0)),
        ],
        out_shape=[
            jax.ShapeDtypeStruct((N_TOK, 1), jnp.int32),
            jax.ShapeDtypeStruct((N_TOK, 1), jnp.int32),
            jax.ShapeDtypeStruct((BLK, 1), jnp.int32),
        ],
        scratch_shapes=[
            pltpu.VMEM((1, E), jnp.float32),
            pltpu.VMEM((1, E), jnp.float32),
        ],
    )(topi, counts)


_TPW = N_TOK // NW


def _k2_body(slots0_hbm, slots1_hbm, xf_hbm, xs_hbm,
             idx0_v, idx1_v, rows_v, sem0, sem1):
    wid = lax.axis_index("s") * 2 + lax.axis_index("c")
    base = wid * _TPW
    pltpu.sync_copy(slots0_hbm.at[pl.ds(base, _TPW)], idx0_v)
    pltpu.sync_copy(slots1_hbm.at[pl.ds(base, _TPW)], idx1_v)
    pltpu.sync_copy(xf_hbm.at[pl.ds(base, _TPW)], rows_v)
    cp0 = pltpu.async_copy(rows_v, xs_hbm.at[idx0_v], sem0)
    cp1 = pltpu.async_copy(rows_v, xs_hbm.at[idx1_v], sem1)
    cp0.wait()
    cp1.wait()


def _scatter_sc(slots0, slots1, xf):
    mesh = plsc.VectorSubcoreMesh(core_axis_name="c", subcore_axis_name="s")
    f = functools.partial(
        pl.kernel,
        out_type=jax.ShapeDtypeStruct((P_ROWS, D), jnp.float32),
        mesh=mesh,
        scratch_types=[
            pltpu.VMEM((_TPW,), jnp.int32),
            pltpu.VMEM((_TPW,), jnp.int32),
            pltpu.VMEM((_TPW, D), jnp.float32),
            pltpu.SemaphoreType.DMA,
            pltpu.SemaphoreType.DMA,
        ],
    )
    return f(_k2_body)(slots0, slots1, xf)


def _k3_body(bem_ref, xs_ref, ew1_ref, ew2_ref, ys_ref):
    h = jnp.dot(xs_ref[...], ew1_ref[0],
                preferred_element_type=jnp.float32)
    h = h * 0.5 * (1.0 + jax.lax.erf(h * (2.0 ** -0.5)))
    ys_ref[...] = jnp.dot(h, ew2_ref[0],
                          preferred_element_type=jnp.float32)


def _expert_ffn(xs, ew1, ew2, bem):
    grid_spec = pltpu.PrefetchScalarGridSpec(
        num_scalar_prefetch=1,
        grid=(NB,),
        in_specs=[
            pl.BlockSpec((BLK, D), lambda b, bem: (b, 0)),
            pl.BlockSpec((1, D, H_ROUTED), lambda b, bem: (bem[b], 0, 0)),
            pl.BlockSpec((1, H_ROUTED, D), lambda b, bem: (bem[b], 0, 0)),
        ],
        out_specs=pl.BlockSpec((BLK, D), lambda b, bem: (b, 0)),
    )
    return pl.pallas_call(
        _k3_body,
        grid_spec=grid_spec,
        out_shape=jax.ShapeDtypeStruct((P_ROWS, D), jnp.float32),
    )(bem, xs, ew1, ew2)


def _k4_body(slots0_hbm, slots1_hbm, ys_hbm, rows0_hbm, rows1_hbm,
             idx0_v, idx1_v, rows0_v, rows1_v, sem0, sem1):
    wid = lax.axis_index("s") * 2 + lax.axis_index("c")
    base = wid * _TPW
    pltpu.sync_copy(slots0_hbm.at[pl.ds(base, _TPW)], idx0_v)
    pltpu.sync_copy(slots1_hbm.at[pl.ds(base, _TPW)], idx1_v)
    cp0 = pltpu.async_copy(ys_hbm.at[idx0_v], rows0_v, sem0)
    cp1 = pltpu.async_copy(ys_hbm.at[idx1_v], rows1_v, sem1)
    cp0.wait()
    pltpu.sync_copy(rows0_v, rows0_hbm.at[pl.ds(base, _TPW)])
    cp1.wait()
    pltpu.sync_copy(rows1_v, rows1_hbm.at[pl.ds(base, _TPW)])


def _gather_sc(slots0, slots1, ys):
    mesh = plsc.VectorSubcoreMesh(core_axis_name="c", subcore_axis_name="s")
    f = functools.partial(
        pl.kernel,
        out_type=[
            jax.ShapeDtypeStruct((N_TOK, D), jnp.float32),
            jax.ShapeDtypeStruct((N_TOK, D), jnp.float32),
        ],
        mesh=mesh,
        scratch_types=[
            pltpu.VMEM((_TPW,), jnp.int32),
            pltpu.VMEM((_TPW,), jnp.int32),
            pltpu.VMEM((_TPW, D), jnp.float32),
            pltpu.VMEM((_TPW, D), jnp.float32),
            pltpu.SemaphoreType.DMA,
            pltpu.SemaphoreType.DMA,
        ],
    )
    return f(_k4_body)(slots0, slots1, ys)


def _k5_body(shared_ref, rows0_ref, rows1_ref, gates_ref, out_ref):
    g = gates_ref[...]
    out_ref[...] = (shared_ref[...] + g[:, 0:1] * rows0_ref[...]
                    + g[:, 1:2] * rows1_ref[...]) * (1.0 / (1 + TOP_K))


def _combine_tc(shared, rows0, rows1, gates):
    grid = N_TOK // BT
    return pl.pallas_call(
        _k5_body,
        grid=(grid,),
        in_specs=[
            pl.BlockSpec((BT, D), lambda i: (i, 0)),
            pl.BlockSpec((BT, D), lambda i: (i, 0)),
            pl.BlockSpec((BT, D), lambda i: (i, 0)),
            pl.BlockSpec((BT, TOP_K), lambda i: (i, 0)),
        ],
        out_specs=pl.BlockSpec((BT, D), lambda i: (i, 0)),
        out_shape=jax.ShapeDtypeStruct((N_TOK, D), jnp.float32),
    )(shared, rows0, rows1, gates)


def kernel(x, t_emb, router_W, sw1, sw3, sw2, ew1, ew2, router_bias):
    B, T, C = x.shape
    xf = x.reshape(-1, C)

    topi, gates, counts = _router(xf, t_emb, router_W, router_bias)
    slots0, slots1, bem = _slot_assign(topi, counts)
    slots0 = slots0.reshape(N_TOK)
    slots1 = slots1.reshape(N_TOK)

    xs = _scatter_sc(slots0, slots1, xf)
    shared = _shared_expert(xf, sw1, sw3, sw2)
    ys = _expert_ffn(xs, ew1, ew2, bem[:NB, 0])
    rows0, rows1 = _gather_sc(slots0, slots1, ys)
    out = _combine_tc(shared, rows0, rows1, gates)
    return out.reshape(B, T, C)


# speedup vs baseline: 1.4108x; 1.0528x over previous
import functools

import jax
import jax.numpy as jnp
from jax import lax
from jax.experimental import pallas as pl
from jax.experimental.pallas import tpu as pltpu
from jax.experimental.pallas import tpu_sc as plsc

E = 8
TOP_K = 2
D = 768
H_SHARED = 1536
H_ROUTED = 768
N_TOK = 2048
N_PAIR = N_TOK * TOP_K
BLK = 256
BLK_SH = BLK.bit_length() - 1
P_ROWS = N_PAIR + E * BLK
NB = P_ROWS // BLK
BT = 256
NW = 32
L = 16

_NT = (((1,), (1,)), ((), ()))


def _k0_body(x_ref, temb_ref, rw_ref, bias_ref,
             gates_ref, slots0_ref, slots1_ref, bem_ref):
    xb = x_ref[...]
    wx = rw_ref[:, :D]
    wt = rw_ref[:, D:]
    logits = (lax.dot_general(xb, wx, _NT,
                              preferred_element_type=jnp.float32)
              + lax.dot_general(temb_ref[...], wt, _NT,
                                preferred_element_type=jnp.float32))
    s = jax.nn.sigmoid(logits)
    sel = s + bias_ref[...]
    idx = lax.broadcasted_iota(jnp.int32, sel.shape, 1)
    neg = jnp.float32(-jnp.inf)
    m1 = jnp.max(sel, axis=1, keepdims=True)
    i1 = jnp.min(jnp.where(sel == m1, idx, E), axis=1, keepdims=True)
    sel2 = jnp.where(idx == i1, neg, sel)
    m2 = jnp.max(sel2, axis=1, keepdims=True)
    i2 = jnp.min(jnp.where(sel2 == m2, idx, E), axis=1, keepdims=True)
    s1 = jnp.sum(jnp.where(idx == i1, s, 0.0), axis=1, keepdims=True)
    s2 = jnp.sum(jnp.where(idx == i2, s, 0.0), axis=1, keepdims=True)
    denom = s1 + s2
    ok = denom > 1e-9
    g1 = jnp.where(ok, s1 / (denom + 1e-9), 1.0 / TOP_K)
    g2 = jnp.where(ok, s2 / (denom + 1e-9), 1.0 / TOP_K)
    gates_ref[:, 0:1] = g1
    gates_ref[:, 1:2] = g2

    oh1 = (idx == i1).astype(jnp.float32)
    oh2 = (idx == i2).astype(jnp.float32)
    S = oh1 + oh2
    cnt = jnp.sum(S, axis=0, keepdims=True).astype(jnp.int32)
    aligned = ((cnt + (BLK - 1)) >> BLK_SH) << BLK_SH
    er = lax.broadcasted_iota(jnp.int32, (E, E), 0)
    ec = lax.broadcasted_iota(jnp.int32, (E, E), 1)
    tri8 = (er < ec).astype(jnp.float32)
    base = jnp.dot(aligned.astype(jnp.float32), tri8,
                   preferred_element_type=jnp.float32)
    bs = lax.broadcasted_iota(jnp.int32, (BLK, E), 0) * BLK
    hits = (bs.astype(jnp.float32) >= base).astype(jnp.float32)
    bem_ref[...] = (jnp.sum(hits, axis=1, keepdims=True)
                    - 1.0).astype(jnp.int32)

    tr = lax.broadcasted_iota(jnp.int32, (BT, BT), 0)
    tc = lax.broadcasted_iota(jnp.int32, (BT, BT), 1)
    tril = (tc < tr).astype(jnp.float32)
    carry = jnp.zeros((1, E), jnp.float32)
    for c in range(N_TOK // BT):
        lo, hi = c * BT, (c + 1) * BT
        Sc = S[lo:hi, :]
        pe = jnp.dot(tril, Sc, preferred_element_type=jnp.float32)
        off = pe + base + carry
        r1 = jnp.sum(jnp.where(oh1[lo:hi, :] > 0, off, 0.0),
                     axis=1, keepdims=True)
        r2 = jnp.sum(jnp.where(oh2[lo:hi, :] > 0, off, 0.0),
                     axis=1, keepdims=True)
        slots0_ref[lo:hi, :] = r1.astype(jnp.int32)
        slots1_ref[lo:hi, :] = r2.astype(jnp.int32)
        carry = carry + jnp.sum(Sc, axis=0, keepdims=True)


def _router(xf, t_emb, router_W, router_bias):
    return pl.pallas_call(
        _k0_body,
        grid=(1,),
        in_specs=[
            pl.BlockSpec((N_TOK, D), lambda i: (0, 0)),
            pl.BlockSpec((1, D), lambda i: (0, 0)),
            pl.BlockSpec((E, 2 * D), lambda i: (0, 0)),
            pl.BlockSpec((1, E), lambda i: (0, 0)),
        ],
        out_specs=[
            pl.BlockSpec((N_TOK, TOP_K), lambda i: (0, 0)),
            pl.BlockSpec((N_TOK, 1), lambda i: (0, 0)),
            pl.BlockSpec((N_TOK, 1), lambda i: (0, 0)),
            pl.BlockSpec((BLK, 1), lambda i: (0, 0)),
        ],
        out_shape=[
            jax.ShapeDtypeStruct((N_TOK, TOP_K), jnp.float32),
            jax.ShapeDtypeStruct((N_TOK, 1), jnp.int32),
            jax.ShapeDtypeStruct((N_TOK, 1), jnp.int32),
            jax.ShapeDtypeStruct((BLK, 1), jnp.int32),
        ],
    )(xf, t_emb, router_W, router_bias.reshape(1, E))


def _k1_body(x_ref, sw1_ref, sw3_ref, sw2_ref, shared_ref):
    xb = x_ref[...]
    a = lax.dot_general(xb, sw1_ref[...], _NT,
                        preferred_element_type=jnp.float32)
    b = lax.dot_general(xb, sw3_ref[...], _NT,
                        preferred_element_type=jnp.float32)
    h = a * jax.nn.sigmoid(a) * b
    shared_ref[...] = lax.dot_general(h, sw2_ref[...], _NT,
                                      preferred_element_type=jnp.float32)


def _shared_expert(xf, sw1, sw3, sw2):
    grid = N_TOK // BT
    return pl.pallas_call(
        _k1_body,
        grid=(grid,),
        in_specs=[
            pl.BlockSpec((BT, D), lambda i: (i, 0)),
            pl.BlockSpec((H_SHARED, D), lambda i: (0, 0)),
            pl.BlockSpec((H_SHARED, D), lambda i: (0, 0)),
            pl.BlockSpec((D, H_SHARED), lambda i: (0, 0)),
        ],
        out_specs=pl.BlockSpec((BT, D), lambda i: (i, 0)),
        out_shape=jax.ShapeDtypeStruct((N_TOK, D), jnp.float32),
    )(xf, sw1, sw3, sw2)


_TPW = N_TOK // NW


def _k2_body(slots0_hbm, slots1_hbm, xf_hbm, xs_hbm,
             idx0_v, idx1_v, rows_v, sem0, sem1):
    wid = lax.axis_index("s") * 2 + lax.axis_index("c")
    base = wid * _TPW
    pltpu.sync_copy(slots0_hbm.at[pl.ds(base, _TPW)], idx0_v)
    pltpu.sync_copy(slots1_hbm.at[pl.ds(base, _TPW)], idx1_v)
    pltpu.sync_copy(xf_hbm.at[pl.ds(base, _TPW)], rows_v)
    cp0 = pltpu.async_copy(rows_v, xs_hbm.at[idx0_v], sem0)
    cp1 = pltpu.async_copy(rows_v, xs_hbm.at[idx1_v], sem1)
    cp0.wait()
    cp1.wait()


def _scatter_sc(slots0, slots1, xf):
    mesh = plsc.VectorSubcoreMesh(core_axis_name="c", subcore_axis_name="s")
    f = functools.partial(
        pl.kernel,
        out_type=jax.ShapeDtypeStruct((P_ROWS, D), jnp.float32),
        mesh=mesh,
        scratch_types=[
            pltpu.VMEM((_TPW,), jnp.int32),
            pltpu.VMEM((_TPW,), jnp.int32),
            pltpu.VMEM((_TPW, D), jnp.float32),
            pltpu.SemaphoreType.DMA,
            pltpu.SemaphoreType.DMA,
        ],
    )
    return f(_k2_body)(slots0, slots1, xf)


def _k3_body(bem_ref, xs_ref, ew1_ref, ew2_ref, ys_ref):
    h = jnp.dot(xs_ref[...], ew1_ref[0],
                preferred_element_type=jnp.float32)
    h = h * 0.5 * (1.0 + jax.lax.erf(h * (2.0 ** -0.5)))
    ys_ref[...] = jnp.dot(h, ew2_ref[0],
                          preferred_element_type=jnp.float32)


def _expert_ffn(xs, ew1, ew2, bem):
    grid_spec = pltpu.PrefetchScalarGridSpec(
        num_scalar_prefetch=1,
        grid=(NB,),
        in_specs=[
            pl.BlockSpec((BLK, D), lambda b, bem: (b, 0)),
            pl.BlockSpec((1, D, H_ROUTED), lambda b, bem: (bem[b], 0, 0)),
            pl.BlockSpec((1, H_ROUTED, D), lambda b, bem: (bem[b], 0, 0)),
        ],
        out_specs=pl.BlockSpec((BLK, D), lambda b, bem: (b, 0)),
    )
    return pl.pallas_call(
        _k3_body,
        grid_spec=grid_spec,
        out_shape=jax.ShapeDtypeStruct((P_ROWS, D), jnp.float32),
    )(bem, xs, ew1, ew2)


def _k4_body(slots0_hbm, slots1_hbm, ys_hbm, rows0_hbm, rows1_hbm,
             idx0_v, idx1_v, rows0_v, rows1_v, sem0, sem1):
    wid = lax.axis_index("s") * 2 + lax.axis_index("c")
    base = wid * _TPW
    pltpu.sync_copy(slots0_hbm.at[pl.ds(base, _TPW)], idx0_v)
    pltpu.sync_copy(slots1_hbm.at[pl.ds(base, _TPW)], idx1_v)
    cp0 = pltpu.async_copy(ys_hbm.at[idx0_v], rows0_v, sem0)
    cp1 = pltpu.async_copy(ys_hbm.at[idx1_v], rows1_v, sem1)
    cp0.wait()
    pltpu.sync_copy(rows0_v, rows0_hbm.at[pl.ds(base, _TPW)])
    cp1.wait()
    pltpu.sync_copy(rows1_v, rows1_hbm.at[pl.ds(base, _TPW)])


def _gather_sc(slots0, slots1, ys):
    mesh = plsc.VectorSubcoreMesh(core_axis_name="c", subcore_axis_name="s")
    f = functools.partial(
        pl.kernel,
        out_type=[
            jax.ShapeDtypeStruct((N_TOK, D), jnp.float32),
            jax.ShapeDtypeStruct((N_TOK, D), jnp.float32),
        ],
        mesh=mesh,
        scratch_types=[
            pltpu.VMEM((_TPW,), jnp.int32),
            pltpu.VMEM((_TPW,), jnp.int32),
            pltpu.VMEM((_TPW, D), jnp.float32),
            pltpu.VMEM((_TPW, D), jnp.float32),
            pltpu.SemaphoreType.DMA,
            pltpu.SemaphoreType.DMA,
        ],
    )
    return f(_k4_body)(slots0, slots1, ys)


def _k5_body(shared_ref, rows0_ref, rows1_ref, gates_ref, out_ref):
    g = gates_ref[...]
    out_ref[...] = (shared_ref[...] + g[:, 0:1] * rows0_ref[...]
                    + g[:, 1:2] * rows1_ref[...]) * (1.0 / (1 + TOP_K))


def _combine_tc(shared, rows0, rows1, gates):
    grid = N_TOK // BT
    return pl.pallas_call(
        _k5_body,
        grid=(grid,),
        in_specs=[
            pl.BlockSpec((BT, D), lambda i: (i, 0)),
            pl.BlockSpec((BT, D), lambda i: (i, 0)),
            pl.BlockSpec((BT, D), lambda i: (i, 0)),
            pl.BlockSpec((BT, TOP_K), lambda i: (i, 0)),
        ],
        out_specs=pl.BlockSpec((BT, D), lambda i: (i, 0)),
        out_shape=jax.ShapeDtypeStruct((N_TOK, D), jnp.float32),
    )(shared, rows0, rows1, gates)


def kernel(x, t_emb, router_W, sw1, sw3, sw2, ew1, ew2, router_bias):
    B, T, C = x.shape
    xf = x.reshape(-1, C)

    gates, slots0, slots1, bem = _router(xf, t_emb, router_W, router_bias)
    slots0 = slots0.reshape(N_TOK)
    slots1 = slots1.reshape(N_TOK)

    xs = _scatter_sc(slots0, slots1, xf)
    shared = _shared_expert(xf, sw1, sw3, sw2)
    ys = _expert_ffn(xs, ew1, ew2, bem[:NB, 0])
    rows0, rows1 = _gather_sc(slots0, slots1, ys)
    out = _combine_tc(shared, rows0, rows1, gates)
    return out.reshape(B, T, C)


# speedup vs baseline: 1.4297x; 1.0134x over previous
import functools

import jax
import jax.numpy as jnp
from jax import lax
from jax.experimental import pallas as pl
from jax.experimental.pallas import tpu as pltpu
from jax.experimental.pallas import tpu_sc as plsc

E = 8
TOP_K = 2
D = 768
H_SHARED = 1536
H_ROUTED = 768
N_TOK = 2048
N_PAIR = N_TOK * TOP_K
BLK = 256
BLK_SH = BLK.bit_length() - 1
P_ROWS = N_PAIR + E * BLK
NB = P_ROWS // BLK
BT = 256
NW = 32
L = 16

_NT = (((1,), (1,)), ((), ()))


def _k0_body(x_ref, temb_ref, rw_ref, bias_ref,
             gates_ref, slots0_ref, slots1_ref, bem_ref):
    xb = x_ref[...]
    wx = rw_ref[:, :D]
    wt = rw_ref[:, D:]
    logits = (lax.dot_general(xb, wx, _NT,
                              preferred_element_type=jnp.float32)
              + lax.dot_general(temb_ref[...], wt, _NT,
                                preferred_element_type=jnp.float32))
    s = jax.nn.sigmoid(logits)
    sel = s + bias_ref[...]
    idx = lax.broadcasted_iota(jnp.int32, sel.shape, 1)
    neg = jnp.float32(-jnp.inf)
    m1 = jnp.max(sel, axis=1, keepdims=True)
    i1 = jnp.min(jnp.where(sel == m1, idx, E), axis=1, keepdims=True)
    sel2 = jnp.where(idx == i1, neg, sel)
    m2 = jnp.max(sel2, axis=1, keepdims=True)
    i2 = jnp.min(jnp.where(sel2 == m2, idx, E), axis=1, keepdims=True)
    s1 = jnp.sum(jnp.where(idx == i1, s, 0.0), axis=1, keepdims=True)
    s2 = jnp.sum(jnp.where(idx == i2, s, 0.0), axis=1, keepdims=True)
    denom = s1 + s2
    ok = denom > 1e-9
    g1 = jnp.where(ok, s1 / (denom + 1e-9), 1.0 / TOP_K)
    g2 = jnp.where(ok, s2 / (denom + 1e-9), 1.0 / TOP_K)
    gates_ref[:, 0:1] = g1
    gates_ref[:, 1:2] = g2

    oh1 = (idx == i1).astype(jnp.float32)
    oh2 = (idx == i2).astype(jnp.float32)
    S = oh1 + oh2
    cnt = jnp.sum(S, axis=0, keepdims=True).astype(jnp.int32)
    aligned = ((cnt + (BLK - 1)) >> BLK_SH) << BLK_SH
    er = lax.broadcasted_iota(jnp.int32, (E, E), 0)
    ec = lax.broadcasted_iota(jnp.int32, (E, E), 1)
    tri8 = (er < ec).astype(jnp.float32)
    base = jnp.dot(aligned.astype(jnp.float32), tri8,
                   preferred_element_type=jnp.float32)
    bs = lax.broadcasted_iota(jnp.int32, (BLK, E), 0) * BLK
    hits = (bs.astype(jnp.float32) >= base).astype(jnp.float32)
    bem_ref[...] = (jnp.sum(hits, axis=1, keepdims=True)
                    - 1.0).astype(jnp.int32)

    tr = lax.broadcasted_iota(jnp.int32, (BT, BT), 0)
    tc = lax.broadcasted_iota(jnp.int32, (BT, BT), 1)
    tril = (tc < tr).astype(jnp.float32)
    carry = jnp.zeros((1, E), jnp.float32)
    for c in range(N_TOK // BT):
        lo, hi = c * BT, (c + 1) * BT
        Sc = S[lo:hi, :]
        pe = jnp.dot(tril, Sc, preferred_element_type=jnp.float32)
        off = pe + base + carry
        r1 = jnp.sum(jnp.where(oh1[lo:hi, :] > 0, off, 0.0),
                     axis=1, keepdims=True)
        r2 = jnp.sum(jnp.where(oh2[lo:hi, :] > 0, off, 0.0),
                     axis=1, keepdims=True)
        slots0_ref[lo:hi, :] = r1.astype(jnp.int32)
        slots1_ref[lo:hi, :] = r2.astype(jnp.int32)
        carry = carry + jnp.sum(Sc, axis=0, keepdims=True)


def _router(xf, t_emb, router_W, router_bias):
    return pl.pallas_call(
        _k0_body,
        grid=(1,),
        in_specs=[
            pl.BlockSpec((N_TOK, D), lambda i: (0, 0)),
            pl.BlockSpec((1, D), lambda i: (0, 0)),
            pl.BlockSpec((E, 2 * D), lambda i: (0, 0)),
            pl.BlockSpec((1, E), lambda i: (0, 0)),
        ],
        out_specs=[
            pl.BlockSpec((N_TOK, TOP_K), lambda i: (0, 0)),
            pl.BlockSpec((N_TOK, 1), lambda i: (0, 0)),
            pl.BlockSpec((N_TOK, 1), lambda i: (0, 0)),
            pl.BlockSpec((BLK, 1), lambda i: (0, 0)),
        ],
        out_shape=[
            jax.ShapeDtypeStruct((N_TOK, TOP_K), jnp.float32),
            jax.ShapeDtypeStruct((N_TOK, 1), jnp.int32),
            jax.ShapeDtypeStruct((N_TOK, 1), jnp.int32),
            jax.ShapeDtypeStruct((BLK, 1), jnp.int32),
        ],
    )(xf, t_emb, router_W, router_bias.reshape(1, E))


def _k1_body(x_ref, sw1_ref, sw3_ref, sw2_ref, shared_ref):
    xb = x_ref[...]
    a = lax.dot_general(xb, sw1_ref[...], _NT,
                        preferred_element_type=jnp.float32)
    b = lax.dot_general(xb, sw3_ref[...], _NT,
                        preferred_element_type=jnp.float32)
    h = a * jax.nn.sigmoid(a) * b
    shared_ref[...] = lax.dot_general(h, sw2_ref[...], _NT,
                                      preferred_element_type=jnp.float32)


def _shared_expert(xf, sw1, sw3, sw2):
    grid = N_TOK // BT
    return pl.pallas_call(
        _k1_body,
        grid=(grid,),
        in_specs=[
            pl.BlockSpec((BT, D), lambda i: (i, 0)),
            pl.BlockSpec((H_SHARED, D), lambda i: (0, 0)),
            pl.BlockSpec((H_SHARED, D), lambda i: (0, 0)),
            pl.BlockSpec((D, H_SHARED), lambda i: (0, 0)),
        ],
        out_specs=pl.BlockSpec((BT, D), lambda i: (i, 0)),
        out_shape=jax.ShapeDtypeStruct((N_TOK, D), jnp.float32),
    )(xf, sw1, sw3, sw2)


_TPW = N_TOK // NW


def _k2_body(slots0_hbm, slots1_hbm, xf_hbm, xs_hbm,
             idx0_v, idx1_v, rows_v, sem0, sem1):
    wid = lax.axis_index("s") * 2 + lax.axis_index("c")
    base = wid * _TPW
    pltpu.sync_copy(slots0_hbm.at[pl.ds(base, _TPW)], idx0_v)
    pltpu.sync_copy(slots1_hbm.at[pl.ds(base, _TPW)], idx1_v)
    pltpu.sync_copy(xf_hbm.at[pl.ds(base, _TPW)], rows_v)
    cp0 = pltpu.async_copy(rows_v, xs_hbm.at[idx0_v], sem0)
    cp1 = pltpu.async_copy(rows_v, xs_hbm.at[idx1_v], sem1)
    cp0.wait()
    cp1.wait()


def _scatter_sc(slots0, slots1, xf):
    mesh = plsc.VectorSubcoreMesh(core_axis_name="c", subcore_axis_name="s")
    f = functools.partial(
        pl.kernel,
        out_type=jax.ShapeDtypeStruct((P_ROWS, D), jnp.float32),
        mesh=mesh,
        scratch_types=[
            pltpu.VMEM((_TPW,), jnp.int32),
            pltpu.VMEM((_TPW,), jnp.int32),
            pltpu.VMEM((_TPW, D), jnp.float32),
            pltpu.SemaphoreType.DMA,
            pltpu.SemaphoreType.DMA,
        ],
    )
    return f(_k2_body)(slots0, slots1, xf)


def _k3_body(bem_ref, xs_ref, ew1_ref, ew2_ref, ys_ref):
    h = jnp.dot(xs_ref[...], ew1_ref[0],
                preferred_element_type=jnp.float32)
    h = h * 0.5 * (1.0 + jax.lax.erf(h * (2.0 ** -0.5)))
    ys_ref[...] = jnp.dot(h, ew2_ref[0],
                          preferred_element_type=jnp.float32)


def _expert_ffn(xs, ew1, ew2, bem):
    grid_spec = pltpu.PrefetchScalarGridSpec(
        num_scalar_prefetch=1,
        grid=(NB,),
        in_specs=[
            pl.BlockSpec((BLK, D), lambda b, bem: (b, 0)),
            pl.BlockSpec((1, D, H_ROUTED), lambda b, bem: (bem[b], 0, 0)),
            pl.BlockSpec((1, H_ROUTED, D), lambda b, bem: (bem[b], 0, 0)),
        ],
        out_specs=pl.BlockSpec((BLK, D), lambda b, bem: (b, 0)),
    )
    return pl.pallas_call(
        _k3_body,
        grid_spec=grid_spec,
        out_shape=jax.ShapeDtypeStruct((P_ROWS, D), jnp.float32),
    )(bem, xs, ew1, ew2)


def _k4_body(slots0_hbm, slots1_hbm, ys_hbm, rows0_hbm, rows1_hbm,
             idx0_v, idx1_v, rows0_v, rows1_v, sem0, sem1):
    wid = lax.axis_index("s") * 2 + lax.axis_index("c")
    base = wid * _TPW
    pltpu.sync_copy(slots0_hbm.at[pl.ds(base, _TPW)], idx0_v)
    pltpu.sync_copy(slots1_hbm.at[pl.ds(base, _TPW)], idx1_v)
    cp0 = pltpu.async_copy(ys_hbm.at[idx0_v], rows0_v, sem0)
    cp1 = pltpu.async_copy(ys_hbm.at[idx1_v], rows1_v, sem1)
    cp0.wait()
    pltpu.sync_copy(rows0_v, rows0_hbm.at[pl.ds(base, _TPW)])
    cp1.wait()
    pltpu.sync_copy(rows1_v, rows1_hbm.at[pl.ds(base, _TPW)])


def _gather_sc(slots0, slots1, ys):
    mesh = plsc.VectorSubcoreMesh(core_axis_name="c", subcore_axis_name="s")
    f = functools.partial(
        pl.kernel,
        out_type=[
            jax.ShapeDtypeStruct((N_TOK, D), jnp.float32),
            jax.ShapeDtypeStruct((N_TOK, D), jnp.float32),
        ],
        mesh=mesh,
        scratch_types=[
            pltpu.VMEM((_TPW,), jnp.int32),
            pltpu.VMEM((_TPW,), jnp.int32),
            pltpu.VMEM((_TPW, D), jnp.float32),
            pltpu.VMEM((_TPW, D), jnp.float32),
            pltpu.SemaphoreType.DMA,
            pltpu.SemaphoreType.DMA,
        ],
    )
    return f(_k4_body)(slots0, slots1, ys)


def _k5_body(shared_ref, rows0_ref, rows1_ref, gates_ref, out_ref):
    g = gates_ref[...]
    out_ref[...] = (shared_ref[...] + g[:, 0:1] * rows0_ref[...]
                    + g[:, 1:2] * rows1_ref[...]) * (1.0 / (1 + TOP_K))


def _combine_tc(shared, rows0, rows1, gates):
    grid = N_TOK // BT
    return pl.pallas_call(
        _k5_body,
        grid=(grid,),
        in_specs=[
            pl.BlockSpec((BT, D), lambda i: (i, 0)),
            pl.BlockSpec((BT, D), lambda i: (i, 0)),
            pl.BlockSpec((BT, D), lambda i: (i, 0)),
            pl.BlockSpec((BT, TOP_K), lambda i: (i, 0)),
        ],
        out_specs=pl.BlockSpec((BT, D), lambda i: (i, 0)),
        out_shape=jax.ShapeDtypeStruct((N_TOK, D), jnp.float32),
    )(shared, rows0, rows1, gates)


def kernel(x, t_emb, router_W, sw1, sw3, sw2, ew1, ew2, router_bias):
    B, T, C = x.shape
    xf = x.reshape(-1, C)

    gates, slots0, slots1, bem = _router(xf, t_emb, router_W, router_bias)
    slots0 = slots0.reshape(N_TOK)
    slots1 = slots1.reshape(N_TOK)

    xs = _scatter_sc(slots0, slots1, xf)
    ys = _expert_ffn(xs, ew1, ew2, bem[:NB, 0])
    rows0, rows1 = _gather_sc(slots0, slots1, ys)
    shared = _shared_expert(xf, sw1, sw3, sw2)
    out = _combine_tc(shared, rows0, rows1, gates)
    return out.reshape(B, T, C)
